# baseline (speedup 1.0000x reference)
import functools

import jax
import jax.numpy as jnp
from jax import lax
from jax.experimental import pallas as pl
from jax.experimental.pallas import tpu as pltpu
from jax.experimental.pallas import tpu_sc as plsc

_B = 8
_N = 256
_D = 128
_L = 16

_NEDGES = 31

_NC = 2
_NS = 16
_NW = _NC * _NS
_RPW = (_B * _N) // _NW
_ROW_W = _N * _D
_UNROLL = 8


def _sc_body(pos_hbm, tab_hbm, thr_hbm, out_hbm, pos_v, thr_v, tab_v,
             buf_a, buf_b, prev_a, prev_b, sem_oa, sem_ob):
    wid = lax.axis_index("s") * _NC + lax.axis_index("c")
    r0g = wid * _RPW
    b = r0g // _N
    i0 = r0g - b * _N
    pltpu.sync_copy(pos_hbm.at[b], pos_v)
    pltpu.sync_copy(thr_hbm, thr_v)
    pltpu.sync_copy(tab_hbm, tab_v)
    lane = lax.broadcasted_iota(jnp.int32, (_L,), 0)
    lane128 = lane * _D
    def do_row(r_loc, buf_ref, prev_ref, sem_o, first):
        i = i0 + r_loc
        @pl.when(jnp.logical_and(jnp.logical_not(first), r_loc < 0))
        def _():
            prev_off = (r0g + r_loc - 2) * _ROW_W
            pltpu.make_async_copy(
                buf_ref, out_hbm.at[pl.ds(prev_off, _ROW_W)], sem_o).wait()
        grp = pos_v[pl.ds((i // _L) * _L, _L)]
        only_i = jnp.where(lane == i % _L, grp, 0)
        pos_i = jnp.full((_L,), jnp.sum(only_i), jnp.int32)
        for g in range(_N // _L):
            pj = pos_v[pl.ds(g * _L, _L)]
            v = jnp.abs(pj - pos_i)
            acc = jnp.zeros((_L,), jnp.int32)
            for k in range(_NEDGES):
                acc = acc + jnp.where(v >= thr_v[k], 1, 0).astype(jnp.int32)
            gbase = acc * _D
            sbase = lane128 + (g * _L * _D)

            @plsc.parallel_loop(0, _L, step=1, unroll=2)
            def _(r):
                rot = jnp.bitwise_and(lane + r, _L - 1)
                groff = gbase + rot
                sroff = sbase + rot
                for m in range(_D // _L):
                    val = plsc.load_gather(tab_v, [groff + (m * _L)])
                    plsc.store_scatter(buf_ref, [sroff + (m * _L)], val)
        @pl.when(r_loc < 0)
        def _():
            pltpu.async_copy(
                buf_ref, out_hbm.at[pl.ds((r0g + r_loc) * _ROW_W, _ROW_W)],
                sem_o)

    def step(t, carry):
        first = t == 0
        do_row(2 * t, buf_a, prev_a, sem_oa, first)
        do_row(2 * t + 1, buf_b, prev_b, sem_ob, first)
        return carry

    lax.fori_loop(0, _RPW // 2, step, 0)
    pltpu.sync_copy(
        buf_a, out_hbm.at[pl.ds((r0g + _RPW - 2) * _ROW_W, _ROW_W)])
    pltpu.sync_copy(
        buf_b, out_hbm.at[pl.ds((r0g + _RPW - 1) * _ROW_W, _ROW_W)])


@jax.jit
def kernel(positions, distance_embed):
    edges = jnp.logspace(0.0, 3.0, _NEDGES, dtype=jnp.float32)
    thr = jnp.floor(edges).astype(jnp.int32) + 1
    thr_b = jnp.broadcast_to(thr[:, None], (_NEDGES, _L))
    tab_flat = distance_embed.reshape(_D * 32)
    mesh = plsc.VectorSubcoreMesh(core_axis_name="c", subcore_axis_name="s")
    run = pl.kernel(
        _sc_body,
        out_type=jax.ShapeDtypeStruct((_B * _N * _N * _D,), jnp.float32),
        mesh=mesh,
        compiler_params=pltpu.CompilerParams(needs_layout_passes=False),
        scratch_types=[
            pltpu.VMEM((_N,), jnp.int32),
            pltpu.VMEM((_NEDGES, _L), jnp.int32),
            pltpu.VMEM((32 * _D,), jnp.float32),
            pltpu.VMEM((_ROW_W,), jnp.float32),
            pltpu.VMEM((_ROW_W,), jnp.float32),
            pltpu.VMEM((_N,), jnp.int32),
            pltpu.VMEM((_N,), jnp.int32),
            pltpu.SemaphoreType.DMA,
            pltpu.SemaphoreType.DMA,
        ],
    )
    out = run(positions, tab_flat, thr_b)
    return out.reshape(_B, _N, _N, _D)


# speedup vs baseline: 2.5572x; 2.5572x over previous
import functools

import jax
import jax.numpy as jnp
from jax import lax
from jax.experimental import pallas as pl
from jax.experimental.pallas import tpu as pltpu
from jax.experimental.pallas import tpu_sc as plsc

_B = 8
_N = 256
_D = 128
_L = 16

_NEDGES = 31

_NC = 2
_NS = 16
_NW = _NC * _NS
_RPW = (_B * _N) // _NW
_ROW_W = _N * _D
_UNROLL = 8


def _sc_body(pos_hbm, tab_hbm, thr_hbm, out_hbm, pos_v, thr_v, tab_v,
             buf_a, buf_b, prev_a, prev_b, wj_a, wb_a, wj_b, wb_b,
             sem_oa, sem_ob):
    wid = lax.axis_index("s") * _NC + lax.axis_index("c")
    r0g = wid * _RPW
    b = r0g // _N
    i0 = r0g - b * _N
    pltpu.sync_copy(pos_hbm.at[b], pos_v)
    pltpu.sync_copy(thr_hbm, thr_v)
    pltpu.sync_copy(tab_hbm, tab_v)
    lane = lax.broadcasted_iota(jnp.int32, (_L,), 0)
    neg1 = jnp.full((_L,), -1, jnp.int32)
    for g in range(_N // _L):
        prev_a[pl.ds(g * _L, _L)] = neg1
        prev_b[pl.ds(g * _L, _L)] = neg1

    def do_row(r_loc, buf_ref, prev_ref, wj_ref, wb_ref, sem_o, first):
        i = i0 + r_loc
        @pl.when(jnp.logical_not(first))
        def _():
            prev_off = (r0g + r_loc - 2) * _ROW_W
            pltpu.make_async_copy(
                buf_ref, out_hbm.at[pl.ds(prev_off, _ROW_W)], sem_o).wait()
        grp = pos_v[pl.ds((i // _L) * _L, _L)]
        only_i = jnp.where(lane == i % _L, grp, 0)
        pos_i = jnp.full((_L,), jnp.sum(only_i), jnp.int32)
        cursor = jnp.int32(0)
        for g in range(_N // _L):
            pj = pos_v[pl.ds(g * _L, _L)]
            v = jnp.abs(pj - pos_i)
            acc = jnp.zeros((_L,), jnp.int32)
            for k in range(_NEDGES):
                acc = acc + jnp.where(v >= thr_v[k], 1, 0).astype(jnp.int32)
            prevb = prev_ref[pl.ds(g * _L, _L)]
            prev_ref[pl.ds(g * _L, _L)] = acc
            chg = (acc != prevb)
            chg_i = jnp.where(chg, 1, 0).astype(jnp.int32)
            pos_c = cursor + plsc.cumsum(chg_i) - chg_i
            plsc.store_scatter(wj_ref, [pos_c], lane + g * _L, mask=chg)
            plsc.store_scatter(wb_ref, [pos_c], acc, mask=chg)
            cursor = cursor + jnp.sum(chg_i)
        nt = (cursor + _L - 1) // _L

        def copy16(t, carry):
            live = (t * _L + lane) < cursor
            jw = jnp.bitwise_and(wj_ref[pl.ds(t * _L, _L)], _N - 1)
            bw = jnp.bitwise_and(wb_ref[pl.ds(t * _L, _L)], 31)
            gbase = bw * _D
            sbase = jw * _D

            @plsc.parallel_loop(0, _L, step=1, unroll=2)
            def _(r):
                rot = jnp.bitwise_and(lane + r, _L - 1)
                groff = gbase + rot
                sroff = sbase + rot
                for m in range(_D // _L):
                    val = plsc.load_gather(
                        tab_v, [groff + (m * _L)], mask=live)
                    plsc.store_scatter(
                        buf_ref, [sroff + (m * _L)], val, mask=live)
            return carry

        lax.fori_loop(0, nt, copy16, 0)
        pltpu.async_copy(
            buf_ref, out_hbm.at[pl.ds((r0g + r_loc) * _ROW_W, _ROW_W)],
            sem_o)

    def step(t, carry):
        first = t == 0
        do_row(2 * t, buf_a, prev_a, wj_a, wb_a, sem_oa, first)
        do_row(2 * t + 1, buf_b, prev_b, wj_b, wb_b, sem_ob, first)
        return carry

    lax.fori_loop(0, _RPW // 2, step, 0)
    pltpu.make_async_copy(
        buf_a, out_hbm.at[pl.ds((r0g + _RPW - 2) * _ROW_W, _ROW_W)],
        sem_oa).wait()
    pltpu.make_async_copy(
        buf_b, out_hbm.at[pl.ds((r0g + _RPW - 1) * _ROW_W, _ROW_W)],
        sem_ob).wait()


@jax.jit
def kernel(positions, distance_embed):
    edges = jnp.logspace(0.0, 3.0, _NEDGES, dtype=jnp.float32)
    thr = jnp.floor(edges).astype(jnp.int32) + 1
    thr_b = jnp.broadcast_to(thr[:, None], (_NEDGES, _L))
    tab_flat = distance_embed.reshape(_D * 32)
    mesh = plsc.VectorSubcoreMesh(core_axis_name="c", subcore_axis_name="s")
    run = pl.kernel(
        _sc_body,
        out_type=jax.ShapeDtypeStruct((_B * _N * _N * _D,), jnp.float32),
        mesh=mesh,
        compiler_params=pltpu.CompilerParams(needs_layout_passes=False),
        scratch_types=[
            pltpu.VMEM((_N,), jnp.int32),
            pltpu.VMEM((_NEDGES, _L), jnp.int32),
            pltpu.VMEM((32 * _D,), jnp.float32),
            pltpu.VMEM((_ROW_W,), jnp.float32),
            pltpu.VMEM((_ROW_W,), jnp.float32),
            pltpu.VMEM((_N,), jnp.int32),
            pltpu.VMEM((_N,), jnp.int32),
            pltpu.VMEM((_N + _L,), jnp.int32),
            pltpu.VMEM((_N + _L,), jnp.int32),
            pltpu.VMEM((_N + _L,), jnp.int32),
            pltpu.VMEM((_N + _L,), jnp.int32),
            pltpu.SemaphoreType.DMA,
            pltpu.SemaphoreType.DMA,
        ],
    )
    out = run(positions, tab_flat, thr_b)
    return out.reshape(_B, _N, _N, _D)
